# baseline (speedup 1.0000x reference)
import functools

import jax
import jax.numpy as jnp
from jax import lax
from jax.experimental import pallas as pl
from jax.experimental.pallas import tpu as pltpu
from jax.experimental.pallas import tpu_sc as plsc

F = 26
V = 100000
E = 32
OUT = 128
B = 16384
EPS = 1e-5

NW = 32
ROWS = B * F
IDX_MINOR = 128
TILE_ROWS = 8
CHUNK = TILE_ROWS * IDX_MINOR
PER_W_TILES = ROWS // IDX_MINOR // NW
STEPS = PER_W_TILES // TILE_ROWS

VB = 1024
NB = 98
VTAIL_B = V - (NB - 1) * VB
G = F * NB
GPW = (G + NW - 1) // NW


def _sc_transpose(src_hbm_arr):
    mesh = plsc.VectorSubcoreMesh(core_axis_name="c", subcore_axis_name="s")

    @functools.partial(
        pl.kernel,
        mesh=mesh,
        out_type=jax.ShapeDtypeStruct((F * V * E,), jnp.float32),
        scratch_types=[
            pltpu.VMEM((E * VB,), jnp.float32),
            pltpu.VMEM((E * VB,), jnp.float32),
        ],
        compiler_params=pltpu.CompilerParams(
            use_tc_tiling_on_sc=False, needs_layout_passes=False),
    )
    def k(src_hbm, scr, src_v, dst_v):
        wid = lax.axis_index("s") * 2 + lax.axis_index("c")
        lanes = lax.iota(jnp.int32, 16)
        pat_lo = lanes * VB
        pat_hi = pat_lo + 16 * VB

        def body(i, carry):
            gid = wid + NW * i

            @pl.when(gid < G)
            def _():
                cb = gid % NB
                f = gid // NB
                v0 = cb * VB
                nv = jnp.where(cb == NB - 1, VTAIL_B, VB)

                @pl.when(cb < NB - 1)
                def _rd_full():
                    for e32 in range(E):
                        pltpu.sync_copy(
                            src_hbm.at[pl.ds((f * E + e32) * V + v0, VB)],
                            src_v.at[pl.ds(e32 * VB, VB)])

                @pl.when(cb == NB - 1)
                def _rd_tail():
                    for e32 in range(E):
                        pltpu.sync_copy(
                            src_hbm.at[pl.ds((f * E + e32) * V + v0, VTAIL_B)],
                            src_v.at[pl.ds(e32 * VB, VTAIL_B)])

                def vloop(v, carry2):
                    lo = plsc.load_gather(src_v, [pat_lo + v])
                    hi = plsc.load_gather(src_v, [pat_hi + v])
                    dst_v[pl.ds(v * E, 16)] = lo
                    dst_v[pl.ds(v * E + 16, 16)] = hi
                    return carry2

                lax.fori_loop(0, nv, vloop, 0)
                o0 = (f * V + v0) * E

                @pl.when(cb < NB - 1)
                def _wf():
                    pltpu.sync_copy(dst_v, scr.at[pl.ds(o0, E * VB)])

                @pl.when(cb == NB - 1)
                def _wt():
                    pltpu.sync_copy(dst_v.at[pl.ds(0, E * VTAIL_B)],
                                    scr.at[pl.ds(o0, E * VTAIL_B)])

            return carry

        lax.fori_loop(0, GPW, body, 0)

    return k(src_hbm_arr)


def _sc_gather(table_flat, idx2):
    mesh = plsc.VectorSubcoreMesh(core_axis_name="c", subcore_axis_name="s")

    @functools.partial(
        pl.kernel,
        mesh=mesh,
        out_type=jax.ShapeDtypeStruct((ROWS, E), jnp.float32),
        scratch_types=[
            pltpu.VMEM((TILE_ROWS, IDX_MINOR), jnp.int32),
            pltpu.VMEM((CHUNK, E), jnp.float32),
            pltpu.SemaphoreType.DMA,
        ],
        compiler_params=pltpu.CompilerParams(use_tc_tiling_on_sc=False),
    )
    def k(tbl, idx_hbm, out_hbm, idx_v, rows_v, sem):
        wid = lax.axis_index("s") * 2 + lax.axis_index("c")
        tile_base = wid * PER_W_TILES

        def body(i, carry):
            t0 = tile_base + i * TILE_ROWS
            pltpu.sync_copy(idx_hbm.at[pl.ds(t0, TILE_ROWS)], idx_v)
            cps = [
                pltpu.async_copy(
                    tbl.at[idx_v.at[j]],
                    rows_v.at[pl.ds(j * IDX_MINOR, IDX_MINOR)],
                    sem,
                )
                for j in range(TILE_ROWS)
            ]
            for cp in cps:
                cp.wait()
            pltpu.sync_copy(rows_v, out_hbm.at[pl.ds(t0 * IDX_MINOR, CHUNK)])
            return carry

        lax.fori_loop(0, STEPS, body, 0)

    return k(table_flat, idx2)


def _tc_proj(emb, W, b, gamma, beta):
    BB = 512

    def body(e_ref, w_ref, b_ref, g_ref, bt_ref, o_ref):
        h = jnp.dot(e_ref[...], w_ref[...], preferred_element_type=jnp.float32)
        h = jnp.maximum(h + b_ref[...], 0.0)
        m = jnp.mean(h, axis=-1, keepdims=True)
        c = h - m
        v = jnp.mean(c * c, axis=-1, keepdims=True)
        o_ref[...] = c * lax.rsqrt(v + EPS) * g_ref[...] + bt_ref[...]

    return pl.pallas_call(
        body,
        grid=(B // BB,),
        in_specs=[
            pl.BlockSpec((BB, F * E), lambda i: (i, 0)),
            pl.BlockSpec((F * E, OUT), lambda i: (0, 0)),
            pl.BlockSpec((1, OUT), lambda i: (0, 0)),
            pl.BlockSpec((1, OUT), lambda i: (0, 0)),
            pl.BlockSpec((1, OUT), lambda i: (0, 0)),
        ],
        out_specs=pl.BlockSpec((BB, OUT), lambda i: (i, 0)),
        out_shape=jax.ShapeDtypeStruct((B, OUT), jnp.float32),
    )(emb, W, b.reshape(1, OUT), gamma.reshape(1, OUT), beta.reshape(1, OUT))


def kernel(x, tables, W, b, gamma, beta):
    t2flat = jnp.swapaxes(tables, 1, 2).reshape(F * E * V)
    scr = _sc_transpose(t2flat)
    table_flat = scr.reshape(F * V, E)

    offs = (jnp.arange(F, dtype=jnp.int32) * V)[None, :]
    idx2 = (x.astype(jnp.int32) + offs).reshape(ROWS // IDX_MINOR, IDX_MINOR)

    emb = _sc_gather(table_flat, idx2)
    return _tc_proj(emb.reshape(B, F * E), W, b, gamma, beta)


# speedup vs baseline: 1.1489x; 1.1489x over previous
import functools

import jax
import jax.numpy as jnp
from jax import lax
from jax.experimental import pallas as pl
from jax.experimental.pallas import tpu as pltpu
from jax.experimental.pallas import tpu_sc as plsc

F = 26
V = 100000
E = 32
OUT = 128
B = 16384
EPS = 1e-5

NW = 32
ROWS = B * F
IDX_MINOR = 128
TILE_ROWS = 8
CHUNK = TILE_ROWS * IDX_MINOR
PER_W_TILES = ROWS // IDX_MINOR // NW
STEPS = PER_W_TILES // TILE_ROWS

VB = 1024
NB = 98
VTAIL_B = V - (NB - 1) * VB
G = F * NB
GPW = (G + NW - 1) // NW


def _sc_transpose(src_hbm_arr):
    mesh = plsc.VectorSubcoreMesh(core_axis_name="c", subcore_axis_name="s")

    @functools.partial(
        pl.kernel,
        mesh=mesh,
        out_type=jax.ShapeDtypeStruct((F * V * E,), jnp.float32),
        scratch_types=[
            pltpu.VMEM((E * VB,), jnp.float32),
            pltpu.VMEM((E * VB,), jnp.float32),
        ],
        compiler_params=pltpu.CompilerParams(
            use_tc_tiling_on_sc=False, needs_layout_passes=False),
    )
    def k(src_hbm, scr, src_v, dst_v):
        wid = lax.axis_index("s") * 2 + lax.axis_index("c")
        lanes = lax.iota(jnp.int32, 16)
        pat_lo = lanes * VB
        pat_hi = pat_lo + 16 * VB

        def body(i, carry):
            gid = wid + NW * i

            @pl.when(gid < G)
            def _():
                cb = gid % NB
                f = gid // NB
                v0 = cb * VB
                nv = jnp.where(cb == NB - 1, VTAIL_B, VB)

                @pl.when(cb < NB - 1)
                def _rd_full():
                    for e32 in range(E):
                        pltpu.sync_copy(
                            src_hbm.at[pl.ds((f * E + e32) * V + v0, VB)],
                            src_v.at[pl.ds(e32 * VB, VB)])

                @pl.when(cb == NB - 1)
                def _rd_tail():
                    for e32 in range(E):
                        pltpu.sync_copy(
                            src_hbm.at[pl.ds((f * E + e32) * V + v0, VTAIL_B)],
                            src_v.at[pl.ds(e32 * VB, VTAIL_B)])

                @plsc.parallel_loop(0, nv, unroll=8)
                def vloop(v):
                    lo = plsc.load_gather(src_v, [pat_lo + v])
                    hi = plsc.load_gather(src_v, [pat_hi + v])
                    dst_v[pl.ds(v * E, 16)] = lo
                    dst_v[pl.ds(v * E + 16, 16)] = hi
                o0 = (f * V + v0) * E

                @pl.when(cb < NB - 1)
                def _wf():
                    pltpu.sync_copy(dst_v, scr.at[pl.ds(o0, E * VB)])

                @pl.when(cb == NB - 1)
                def _wt():
                    pltpu.sync_copy(dst_v.at[pl.ds(0, E * VTAIL_B)],
                                    scr.at[pl.ds(o0, E * VTAIL_B)])

            return carry

        lax.fori_loop(0, GPW, body, 0)

    return k(src_hbm_arr)


def _sc_gather(table_flat, idx2):
    mesh = plsc.VectorSubcoreMesh(core_axis_name="c", subcore_axis_name="s")

    @functools.partial(
        pl.kernel,
        mesh=mesh,
        out_type=jax.ShapeDtypeStruct((ROWS, E), jnp.float32),
        scratch_types=[
            pltpu.VMEM((TILE_ROWS, IDX_MINOR), jnp.int32),
            pltpu.VMEM((CHUNK, E), jnp.float32),
            pltpu.SemaphoreType.DMA,
        ],
        compiler_params=pltpu.CompilerParams(use_tc_tiling_on_sc=False),
    )
    def k(tbl, idx_hbm, out_hbm, idx_v, rows_v, sem):
        wid = lax.axis_index("s") * 2 + lax.axis_index("c")
        tile_base = wid * PER_W_TILES

        def body(i, carry):
            t0 = tile_base + i * TILE_ROWS
            pltpu.sync_copy(idx_hbm.at[pl.ds(t0, TILE_ROWS)], idx_v)
            cps = [
                pltpu.async_copy(
                    tbl.at[idx_v.at[j]],
                    rows_v.at[pl.ds(j * IDX_MINOR, IDX_MINOR)],
                    sem,
                )
                for j in range(TILE_ROWS)
            ]
            for cp in cps:
                cp.wait()
            pltpu.sync_copy(rows_v, out_hbm.at[pl.ds(t0 * IDX_MINOR, CHUNK)])
            return carry

        lax.fori_loop(0, STEPS, body, 0)

    return k(table_flat, idx2)


def _tc_proj(emb, W, b, gamma, beta):
    BB = 512

    def body(e_ref, w_ref, b_ref, g_ref, bt_ref, o_ref):
        h = jnp.dot(e_ref[...], w_ref[...], preferred_element_type=jnp.float32)
        h = jnp.maximum(h + b_ref[...], 0.0)
        m = jnp.mean(h, axis=-1, keepdims=True)
        c = h - m
        v = jnp.mean(c * c, axis=-1, keepdims=True)
        o_ref[...] = c * lax.rsqrt(v + EPS) * g_ref[...] + bt_ref[...]

    return pl.pallas_call(
        body,
        grid=(B // BB,),
        in_specs=[
            pl.BlockSpec((BB, F * E), lambda i: (i, 0)),
            pl.BlockSpec((F * E, OUT), lambda i: (0, 0)),
            pl.BlockSpec((1, OUT), lambda i: (0, 0)),
            pl.BlockSpec((1, OUT), lambda i: (0, 0)),
            pl.BlockSpec((1, OUT), lambda i: (0, 0)),
        ],
        out_specs=pl.BlockSpec((BB, OUT), lambda i: (i, 0)),
        out_shape=jax.ShapeDtypeStruct((B, OUT), jnp.float32),
    )(emb, W, b.reshape(1, OUT), gamma.reshape(1, OUT), beta.reshape(1, OUT))


def kernel(x, tables, W, b, gamma, beta):
    t2flat = jnp.swapaxes(tables, 1, 2).reshape(F * E * V)
    scr = _sc_transpose(t2flat)
    table_flat = scr.reshape(F * V, E)

    offs = (jnp.arange(F, dtype=jnp.int32) * V)[None, :]
    idx2 = (x.astype(jnp.int32) + offs).reshape(ROWS // IDX_MINOR, IDX_MINOR)

    emb = _sc_gather(table_flat, idx2)
    return _tc_proj(emb.reshape(B, F * E), W, b, gamma, beta)
